# baseline (speedup 1.0000x reference)
import functools

import jax
import jax.numpy as jnp
from jax.experimental import pallas as pl
from jax.experimental.pallas import tpu as pltpu


def _fused_kernel(x_ref, gW1_ref, gb1_ref, gln_g_ref, gln_b_ref, gW2_ref,
                  gb2_ref, eW_ref, eb_ref, cW1_ref, cb1_ref, cln_g_ref,
                  cln_b_ref, cW2_ref, cb2_ref, logits_ref, probs_ref,
                  acc_ref, comb_ref, *, E):
    e = pl.program_id(0)
    B = x_ref.shape[0]

    @pl.when(e == 0)
    def _gating():
        x = x_ref[...]
        h = jnp.dot(x, gW1_ref[...], preferred_element_type=jnp.float32)
        h = h + gb1_ref[0]
        mu = jnp.mean(h, axis=-1, keepdims=True)
        var = jnp.mean((h - mu) ** 2, axis=-1, keepdims=True)
        h = (h - mu) * jax.lax.rsqrt(var + 1e-5) * gln_g_ref[0] + gln_b_ref[0]
        h = jax.nn.gelu(h)
        glog = jnp.dot(h, gW2_ref[...], preferred_element_type=jnp.float32)
        glog = glog + gb2_ref[0]
        probs = jax.nn.softmax(glog, axis=-1)
        probs_ref[...] = probs

        e_iota = jax.lax.broadcasted_iota(jnp.int32, (B, E), 1)
        v1 = jnp.max(probs, axis=-1, keepdims=True)
        i1 = jnp.min(jnp.where(probs == v1, e_iota, E), axis=-1, keepdims=True)
        mask1 = e_iota == i1
        probs_rest = jnp.where(mask1, -1.0, probs)
        v2 = jnp.max(probs_rest, axis=-1, keepdims=True)
        i2 = jnp.min(jnp.where(probs_rest == v2, e_iota, E), axis=-1,
                     keepdims=True)
        mask2 = e_iota == i2
        comb_ref[...] = (v1 * mask1.astype(jnp.float32)
                         + v2 * mask2.astype(jnp.float32))

    comb = comb_ref[...]
    ce = jnp.sum(jnp.where(
        jax.lax.broadcasted_iota(jnp.int32, comb.shape, 1) == e, comb, 0.0),
        axis=1, keepdims=True)
    contrib = ce * jnp.dot(
        x_ref[...], eW_ref[0].astype(jnp.float32),
        preferred_element_type=jnp.float32)

    @pl.when(e == 0)
    def _init():
        acc_ref[...] = contrib

    @pl.when(e > 0)
    def _accum():
        acc_ref[...] += contrib

    @pl.when(e == E - 1)
    def _classifier():
        ci = acc_ref[...] + jnp.dot(comb_ref[...], eb_ref[...],
                                    preferred_element_type=jnp.float32)
        ch = jnp.dot(ci, cW1_ref[...], preferred_element_type=jnp.float32)
        ch = ch + cb1_ref[0]
        mu = jnp.mean(ch, axis=-1, keepdims=True)
        var = jnp.mean((ch - mu) ** 2, axis=-1, keepdims=True)
        ch = ((ch - mu) * jax.lax.rsqrt(var + 1e-5) * cln_g_ref[0]
              + cln_b_ref[0])
        ch = jnp.maximum(ch, 0.0)
        logits = jnp.dot(ch, cW2_ref[...], preferred_element_type=jnp.float32)
        logits_ref[...] = logits + cb2_ref[0]


def kernel(x, gW1, gb1, gln_g, gln_b, gW2, gb2, eW, eb, cW1, cb1, cln_g,
           cln_b, cW2, cb2):
    B, H = x.shape
    E = eW.shape[0]
    C = cW2.shape[1]

    def row(v):
        return v.reshape(1, -1)

    full = lambda a: pl.BlockSpec(a.shape, lambda i: (0,) * a.ndim)
    out = pl.pallas_call(
        functools.partial(_fused_kernel, E=E),
        grid=(E,),
        in_specs=[
            full(x),
            full(gW1), full(row(gb1)), full(row(gln_g)), full(row(gln_b)),
            full(gW2), full(row(gb2)),
            pl.BlockSpec((1, H, H), lambda i: (i, 0, 0)),
            full(eb),
            full(cW1), full(row(cb1)), full(row(cln_g)), full(row(cln_b)),
            full(cW2), full(row(cb2)),
        ],
        out_specs=[
            pl.BlockSpec((B, C), lambda i: (0, 0)),
            pl.BlockSpec((B, E), lambda i: (0, 0)),
        ],
        out_shape=[
            jax.ShapeDtypeStruct((B, C), jnp.float32),
            jax.ShapeDtypeStruct((B, E), jnp.float32),
        ],
        scratch_shapes=[
            pltpu.VMEM((B, H), jnp.float32),
            pltpu.VMEM((B, E), jnp.float32),
        ],
    )(x, gW1, row(gb1), row(gln_g), row(gln_b), gW2, row(gb2),
      eW.astype(jnp.bfloat16), eb,
      cW1, row(cb1), row(cln_g), row(cln_b), cW2, row(cb2))
    return out[0], out[1]


# speedup vs baseline: 1.2330x; 1.2330x over previous
import functools

import jax
import jax.numpy as jnp
from jax.experimental import pallas as pl
from jax.experimental.pallas import tpu as pltpu


def _fused_kernel(x_ref, gW1_ref, gb1_ref, gln_g_ref, gln_b_ref, gW2_ref,
                  gb2_ref, eW_ref, eb_ref, cW1_ref, cb1_ref, cln_g_ref,
                  cln_b_ref, cW2_ref, cb2_ref, logits_ref, probs_ref,
                  acc_ref, comb_ref, *, E):
    e = pl.program_id(0)
    B = x_ref.shape[0]

    @pl.when(e == 0)
    def _gating():
        x = x_ref[...]
        h = jnp.dot(x, gW1_ref[...], preferred_element_type=jnp.float32)
        h = h + gb1_ref[0]
        mu = jnp.mean(h, axis=-1, keepdims=True)
        var = jnp.mean((h - mu) ** 2, axis=-1, keepdims=True)
        h = (h - mu) * jax.lax.rsqrt(var + 1e-5) * gln_g_ref[0] + gln_b_ref[0]
        h = jax.nn.gelu(h)
        glog = jnp.dot(h, gW2_ref[...], preferred_element_type=jnp.float32)
        glog = glog + gb2_ref[0]
        probs = jax.nn.softmax(glog, axis=-1)
        probs_ref[...] = probs

        e_iota = jax.lax.broadcasted_iota(jnp.int32, (B, E), 1)
        v1 = jnp.max(probs, axis=-1, keepdims=True)
        i1 = jnp.min(jnp.where(probs == v1, e_iota, E), axis=-1, keepdims=True)
        mask1 = e_iota == i1
        probs_rest = jnp.where(mask1, -1.0, probs)
        v2 = jnp.max(probs_rest, axis=-1, keepdims=True)
        i2 = jnp.min(jnp.where(probs_rest == v2, e_iota, E), axis=-1,
                     keepdims=True)
        mask2 = e_iota == i2
        comb_ref[...] = (v1 * mask1.astype(jnp.float32)
                         + v2 * mask2.astype(jnp.float32))

    comb = comb_ref[...]
    ce = jnp.sum(jnp.where(
        jax.lax.broadcasted_iota(jnp.int32, comb.shape, 1) == e, comb, 0.0),
        axis=1, keepdims=True)
    contrib = ce * jnp.dot(
        x_ref[...], eW_ref[0], preferred_element_type=jnp.float32)

    @pl.when(e == 0)
    def _init():
        acc_ref[...] = contrib

    @pl.when(e > 0)
    def _accum():
        acc_ref[...] += contrib

    @pl.when(e == E - 1)
    def _classifier():
        ci = acc_ref[...] + jnp.dot(comb_ref[...], eb_ref[...],
                                    preferred_element_type=jnp.float32)
        ch = jnp.dot(ci, cW1_ref[...], preferred_element_type=jnp.float32)
        ch = ch + cb1_ref[0]
        mu = jnp.mean(ch, axis=-1, keepdims=True)
        var = jnp.mean((ch - mu) ** 2, axis=-1, keepdims=True)
        ch = ((ch - mu) * jax.lax.rsqrt(var + 1e-5) * cln_g_ref[0]
              + cln_b_ref[0])
        ch = jnp.maximum(ch, 0.0)
        logits = jnp.dot(ch, cW2_ref[...], preferred_element_type=jnp.float32)
        logits_ref[...] = logits + cb2_ref[0]


def kernel(x, gW1, gb1, gln_g, gln_b, gW2, gb2, eW, eb, cW1, cb1, cln_g,
           cln_b, cW2, cb2):
    B, H = x.shape
    E = eW.shape[0]
    C = cW2.shape[1]

    def row(v):
        return v.reshape(1, -1)

    full = lambda a: pl.BlockSpec(a.shape, lambda i: (0,) * a.ndim)
    out = pl.pallas_call(
        functools.partial(_fused_kernel, E=E),
        grid=(E,),
        in_specs=[
            full(x),
            full(gW1), full(row(gb1)), full(row(gln_g)), full(row(gln_b)),
            full(gW2), full(row(gb2)),
            pl.BlockSpec((1, H, H), lambda i: (i, 0, 0)),
            full(eb),
            full(cW1), full(row(cb1)), full(row(cln_g)), full(row(cln_b)),
            full(cW2), full(row(cb2)),
        ],
        out_specs=[
            pl.BlockSpec((B, C), lambda i: (0, 0)),
            pl.BlockSpec((B, E), lambda i: (0, 0)),
        ],
        out_shape=[
            jax.ShapeDtypeStruct((B, C), jnp.float32),
            jax.ShapeDtypeStruct((B, E), jnp.float32),
        ],
        scratch_shapes=[
            pltpu.VMEM((B, H), jnp.float32),
            pltpu.VMEM((B, E), jnp.float32),
        ],
    )(x, gW1, row(gb1), row(gln_g), row(gln_b), gW2, row(gb2), eW, eb,
      cW1, row(cb1), row(cln_g), row(cln_b), cW2, row(cb2))
    return out[0], out[1]
